# baseline (speedup 1.0000x reference)
import functools

import jax
import jax.numpy as jnp
from jax import lax
from jax.experimental import pallas as pl
from jax.experimental.pallas import tpu as pltpu
from jax.experimental.pallas import tpu_sc as plsc

V = 500
D = 12
NP = D // 2
VP = 512
B = 16384
BAG = 200
NC, NS, LANES = 2, 16, 16
NW = NC * NS
BPW = B // NW
CH = 64
NCHUNK = BPW // CH
GPC = CH // LANES
NBLK = BAG // 8

_mesh = plsc.VectorSubcoreMesh(core_axis_name="c", subcore_axis_name="s")


@functools.partial(
    pl.kernel,
    out_type=jax.ShapeDtypeStruct((B * D,), jnp.float32),
    mesh=_mesh,
    compiler_params=pltpu.CompilerParams(needs_layout_passes=False),
    scratch_types=[
        pltpu.VMEM((NP * VP * LANES,), jnp.int32),
        pltpu.VMEM((CH, BAG), jnp.int32),
        pltpu.VMEM((CH, BAG), jnp.int32),
        pltpu.VMEM((CH * D,), jnp.float32),
        pltpu.SemaphoreType.DMA,
        pltpu.SemaphoreType.DMA,
    ],
)
def _emb_bag(tab_hbm, idx_hbm, out_hbm, tab_v, idx_a, idx_b, out_v,
             sem_a, sem_b):
    wid = lax.axis_index("s") * NC + lax.axis_index("c")
    base_bag = wid * BPW
    pltpu.sync_copy(tab_hbm, tab_v)

    bufs = [(idx_a, sem_a), (idx_b, sem_b)]

    def start(c):
        buf, sem = bufs[c % 2]
        return pltpu.async_copy(
            idx_hbm.at[pl.ds(base_bag + c * CH, CH), :], buf, sem)

    pending = {0: start(0)}
    lane = lax.iota(jnp.int32, LANES)
    lane_bag = lane * BAG
    lane_out = lane * D
    inv = jnp.float32(1.0 / BAG)
    tab_p = [tab_v.at[pl.ds(p * VP * LANES, VP * LANES)] for p in range(NP)]

    for c in range(NCHUNK):
        if c + 1 < NCHUNK:
            pending[c + 1] = start(c + 1)
        pending.pop(c).wait()
        buf = bufs[c % 2][0]
        for g in range(GPC):
            bag_sel = lane + g * LANES

            def lbody(j, accs, buf=buf, bag_sel=bag_sel):
                accs = list(accs)
                bf = [jnp.zeros((2 * LANES,), jnp.bfloat16) for _ in range(NP)]
                for k in range(8):
                    pos = jnp.zeros((LANES,), jnp.int32) + (j * 8 + k)
                    rows = plsc.load_gather(buf, [bag_sel, pos])
                    rs = rows * LANES + lane
                    for p in range(NP):
                        word = plsc.load_gather(tab_p[p], [rs])
                        bf[p] = bf[p] + plsc.bitcast(word, jnp.bfloat16)
                for p in range(NP):
                    a, b = plsc.unpack(bf[p], format=plsc.PackFormat.INTERLEAVED)
                    accs[2 * p] = accs[2 * p] + a
                    accs[2 * p + 1] = accs[2 * p + 1] + b
                return tuple(accs)

            accs = lax.fori_loop(
                0, NBLK, lbody,
                tuple(jnp.zeros((LANES,), jnp.float32) for _ in range(D)))
            for d in range(D):
                plsc.store_scatter(out_v, [lane_out + (g * LANES * D + d)],
                                   accs[d] * inv)
        pltpu.sync_copy(out_v,
                        out_hbm.at[pl.ds((base_bag + c * CH) * D, CH * D)])


def kernel(x_user, weight):
    wb = jnp.pad(weight.astype(jnp.bfloat16), ((0, VP - V), (0, 0)))
    wpair = lax.bitcast_convert_type(wb.reshape(VP, NP, 2), jnp.int32)
    wrep = jnp.broadcast_to(wpair.T[:, :, None], (NP, VP, LANES)).reshape(-1)
    out = _emb_bag(wrep, x_user)
    return out.reshape(B, D)


# speedup vs baseline: 1.5762x; 1.5762x over previous
import functools

import jax
import jax.numpy as jnp
from jax import lax
from jax.experimental import pallas as pl
from jax.experimental.pallas import tpu as pltpu
from jax.experimental.pallas import tpu_sc as plsc

V = 500
D = 12
NP = D // 2
VP = 512
B = 16384
BAG = 200
W100 = BAG // 2
NC, NS, LANES = 2, 16, 16
NW = NC * NS
BPW = B // NW
CH = 128
NCHUNK = BPW // CH
GPC = CH // LANES
NBLK = W100 // 4

_mesh = plsc.VectorSubcoreMesh(core_axis_name="c", subcore_axis_name="s")


@functools.partial(
    pl.kernel,
    out_type=jax.ShapeDtypeStruct((B * D,), jnp.float32),
    mesh=_mesh,
    compiler_params=pltpu.CompilerParams(needs_layout_passes=False),
    scratch_types=[
        pltpu.VMEM((NP * VP * LANES,), jnp.int32),
        pltpu.VMEM((W100, CH), jnp.int32),
        pltpu.VMEM((W100, CH), jnp.int32),
        pltpu.VMEM((CH * D,), jnp.float32),
        pltpu.SemaphoreType.DMA,
        pltpu.SemaphoreType.DMA,
    ],
)
def _emb_bag(tab_hbm, idx_hbm, out_hbm, tab_v, idx_a, idx_b, out_v,
             sem_a, sem_b):
    wid = lax.axis_index("s") * NC + lax.axis_index("c")
    base_bag = wid * BPW
    pltpu.sync_copy(tab_hbm, tab_v)

    bufs = [(idx_a, sem_a), (idx_b, sem_b)]

    def start(c):
        buf, sem = bufs[c % 2]
        return pltpu.async_copy(
            idx_hbm.at[:, pl.ds(base_bag + c * CH, CH)], buf, sem)

    pending = {0: start(0)}
    lane = lax.iota(jnp.int32, LANES)
    lane_out = lane * D
    inv = jnp.float32(1.0 / BAG)
    tab_p = [tab_v.at[pl.ds(p * VP * LANES, VP * LANES)] for p in range(NP)]

    for c in range(NCHUNK):
        if c + 1 < NCHUNK:
            pending[c + 1] = start(c + 1)
        pending.pop(c).wait()
        buf = bufs[c % 2][0]
        for g in range(GPC):

            def lbody(j, accs, buf=buf, g=g):
                accs = list(accs)
                bf = [jnp.zeros((2 * LANES,), jnp.bfloat16) for _ in range(NP)]
                for k in range(4):
                    w = buf[j * 4 + k, pl.ds(g * LANES, LANES)]
                    rlo = (w & 0xFFFF) * LANES + lane
                    rhi = lax.shift_right_logical(w, 16) * LANES + lane
                    for rs in (rlo, rhi):
                        for p in range(NP):
                            word = plsc.load_gather(tab_p[p], [rs])
                            bf[p] = bf[p] + plsc.bitcast(word, jnp.bfloat16)
                for p in range(NP):
                    a, b = plsc.unpack(bf[p], format=plsc.PackFormat.INTERLEAVED)
                    accs[2 * p] = accs[2 * p] + a
                    accs[2 * p + 1] = accs[2 * p + 1] + b
                return tuple(accs)

            accs = lax.fori_loop(
                0, NBLK, lbody,
                tuple(jnp.zeros((LANES,), jnp.float32) for _ in range(D)))
            for d in range(D):
                plsc.store_scatter(out_v, [lane_out + (g * LANES * D + d)],
                                   accs[d] * inv)
        pltpu.sync_copy(out_v,
                        out_hbm.at[pl.ds((base_bag + c * CH) * D, CH * D)])


def kernel(x_user, weight):
    xt = jnp.bitwise_or(x_user[:, :W100],
                        jnp.left_shift(x_user[:, W100:], 16)).T
    wb = jnp.pad(weight.astype(jnp.bfloat16), ((0, VP - V), (0, 0)))
    wpair = lax.bitcast_convert_type(wb.reshape(VP, NP, 2), jnp.int32)
    wrep = jnp.broadcast_to(wpair.T[:, :, None], (NP, VP, LANES)).reshape(-1)
    out = _emb_bag(wrep, xt)
    return out.reshape(B, D)


# speedup vs baseline: 1.5807x; 1.0028x over previous
import functools

import jax
import jax.numpy as jnp
from jax import lax
from jax.experimental import pallas as pl
from jax.experimental.pallas import tpu as pltpu
from jax.experimental.pallas import tpu_sc as plsc

V = 500
D = 12
NP = D // 2
VP = 512
B = 16384
BAG = 200
W100 = BAG // 2
NC, NS, LANES = 2, 16, 16
NW = NC * NS
BPW = B // NW
CH = 256
NCHUNK = BPW // CH
GPC = CH // LANES
NBLK = W100 // 4

_mesh = plsc.VectorSubcoreMesh(core_axis_name="c", subcore_axis_name="s")


@functools.partial(
    pl.kernel,
    out_type=jax.ShapeDtypeStruct((B * D,), jnp.float32),
    mesh=_mesh,
    compiler_params=pltpu.CompilerParams(needs_layout_passes=False),
    scratch_types=[
        pltpu.VMEM((NP * VP * LANES,), jnp.int32),
        pltpu.VMEM((W100, CH), jnp.int32),
        pltpu.VMEM((W100, CH), jnp.int32),
        pltpu.VMEM((CH * D,), jnp.float32),
        pltpu.VMEM((CH * D,), jnp.float32),
        pltpu.SemaphoreType.DMA,
        pltpu.SemaphoreType.DMA,
        pltpu.SemaphoreType.DMA,
    ],
)
def _emb_bag(tab_hbm, idx_hbm, out_hbm, tab_v, idx_a, idx_b, out_a, out_b,
             sem_a, sem_b, sem_o):
    wid = lax.axis_index("s") * NC + lax.axis_index("c")
    base_bag = wid * BPW

    bufs = [(idx_a, sem_a), (idx_b, sem_b)]
    obufs = [out_a, out_b]

    def start(c):
        buf, sem = bufs[c % 2]
        return pltpu.async_copy(
            idx_hbm.at[:, pl.ds(base_bag + c * CH, CH)], buf, sem)

    pending = {0: start(0)}
    pltpu.sync_copy(tab_hbm, tab_v)
    lane = lax.iota(jnp.int32, LANES)
    lane_out = lane * D
    inv = jnp.float32(1.0 / BAG)
    tab_p = [tab_v.at[pl.ds(p * VP * LANES, VP * LANES)] for p in range(NP)]

    out_pending = []
    for c in range(NCHUNK):
        if c + 1 < NCHUNK:
            pending[c + 1] = start(c + 1)
        pending.pop(c).wait()
        buf = bufs[c % 2][0]
        out_v = obufs[c % 2]
        for g in range(GPC):

            def lbody(j, accs, buf=buf, g=g):
                accs = list(accs)
                bf = [jnp.zeros((2 * LANES,), jnp.bfloat16) for _ in range(NP)]
                for k in range(4):
                    w = buf[j * 4 + k, pl.ds(g * LANES, LANES)]
                    rlo = (w & 0xFFFF) * LANES + lane
                    rhi = lax.shift_right_logical(w, 16) * LANES + lane
                    for rs in (rlo, rhi):
                        for p in range(NP):
                            word = plsc.load_gather(tab_p[p], [rs])
                            bf[p] = bf[p] + plsc.bitcast(word, jnp.bfloat16)
                for p in range(NP):
                    a, b = plsc.unpack(bf[p], format=plsc.PackFormat.INTERLEAVED)
                    accs[2 * p] = accs[2 * p] + a
                    accs[2 * p + 1] = accs[2 * p + 1] + b
                return tuple(accs)

            accs = lax.fori_loop(
                0, NBLK, lbody,
                tuple(jnp.zeros((LANES,), jnp.float32) for _ in range(D)))
            for d in range(D):
                plsc.store_scatter(out_v, [lane_out + (g * LANES * D + d)],
                                   accs[d] * inv)
        out_pending.append(pltpu.async_copy(
            out_v, out_hbm.at[pl.ds((base_bag + c * CH) * D, CH * D)], sem_o))
    for cp in out_pending:
        cp.wait()


def kernel(x_user, weight):
    xt = jnp.bitwise_or(x_user[:, :W100],
                        jnp.left_shift(x_user[:, W100:], 16)).T
    wb = jnp.pad(weight.astype(jnp.bfloat16), ((0, VP - V), (0, 0)))
    wpair = lax.bitcast_convert_type(wb.reshape(VP, NP, 2), jnp.int32)
    wrep = jnp.broadcast_to(wpair.T[:, :, None], (NP, VP, LANES)).reshape(-1)
    out = _emb_bag(wrep, xt)
    return out.reshape(B, D)
